# baseline (speedup 1.0000x reference)
import functools

import jax
import jax.numpy as jnp
from jax import lax
from jax.experimental import pallas as pl
from jax.experimental.pallas import tpu as pltpu
from jax.experimental.pallas import tpu_sc as plsc

EMB = 512
OUT = 512
NW = 32
PIECE = 128
NPC = EMB // PIECE
CHP = 128
PCH = 120
M_BLK = 1024


def _make_sc_gather(b_total):
  b_per_w = b_total // NW
  p_per_w = b_per_w * NPC
  nchunk = p_per_w // CHP
  npos = b_per_w // PCH
  assert b_per_w % PCH == 0 and p_per_w % CHP == 0 and nchunk % 2 == 0

  mesh = plsc.VectorSubcoreMesh(core_axis_name="c", subcore_axis_name="s")

  @functools.partial(
      pl.kernel,
      mesh=mesh,
      out_type=[
          jax.ShapeDtypeStruct((b_total * NPC, PIECE), jnp.float32),
          jax.ShapeDtypeStruct((b_total,), jnp.int32),
      ],
      scratch_types=[
          pltpu.VMEM((b_per_w,), jnp.int32),
          pltpu.VMEM((p_per_w,), jnp.int32),
          pltpu.VMEM((b_per_w,), jnp.int32),
          pltpu.VMEM((2, CHP, PIECE), jnp.float32),
          pltpu.SemaphoreType.DMA,
          pltpu.SemaphoreType.DMA,
          pltpu.SemaphoreType.DMA,
      ],
  )
  def gather_kernel(table4_hbm, posmap_hbm, ids_hbm, idx4_hbm,
                    pieces_out_hbm, pos_out_hbm,
                    ids_v, idx4_v, pos_v, ring_v, gsem, psem, osem):
    wid = lax.axis_index("s") * 2 + lax.axis_index("c")
    base = wid * b_per_w
    pbase = wid * p_per_w
    pltpu.sync_copy(ids_hbm.at[pl.ds(base, b_per_w)], ids_v)
    pltpu.sync_copy(idx4_hbm.at[pl.ds(pbase, p_per_w)], idx4_v)

    pos_copies = []
    for c in range(npos):
      sl = pl.ds(c * PCH, PCH)
      pos_copies.append(
          pltpu.async_copy(posmap_hbm.at[ids_v.at[sl]], pos_v.at[sl], psem))

    def chunk(c, slot, drain):
      if drain:
        pltpu.make_async_copy(
            ring_v.at[slot], pieces_out_hbm.at[pl.ds(pbase, CHP)], osem).wait()
      g = pltpu.async_copy(
          table4_hbm.at[idx4_v.at[pl.ds(c * CHP, CHP)]], ring_v.at[slot], gsem)
      g.wait()
      pltpu.async_copy(
          ring_v.at[slot], pieces_out_hbm.at[pl.ds(pbase + c * CHP, CHP)], osem)

    chunk(0, 0, False)
    chunk(1, 1, False)

    def body(g, _):
      chunk(2 * g, 0, True)
      chunk(2 * g + 1, 1, True)
      return _

    lax.fori_loop(1, nchunk // 2, body, 0)

    for slot in (0, 1):
      pltpu.make_async_copy(
          ring_v.at[slot], pieces_out_hbm.at[pl.ds(pbase, CHP)], osem).wait()
    for pc in pos_copies:
      pc.wait()
    pltpu.sync_copy(pos_v, pos_out_hbm.at[pl.ds(base, b_per_w)])

  return gather_kernel


def _project_block(x_ref, pidx_ref, poshead_ref, we_ref, wp_ref, b_ref, o_ref):
  acc = b_ref[...] + jnp.zeros((M_BLK, OUT), jnp.float32)
  for j in range(NPC):
    xj = x_ref[:, j].reshape(M_BLK, PIECE)
    acc = acc + lax.dot_general(xj, we_ref[:, j], (((1,), (1,)), ((), ())),
                                preferred_element_type=jnp.float32)
  pidx = pidx_ref[0, 0, :]
  ph = (pidx[:, None] == lax.broadcasted_iota(jnp.int32, (M_BLK, 16), 1)
        ).astype(jnp.float32)
  pp = lax.dot_general(poshead_ref[...], wp_ref[...],
                       (((1,), (1,)), ((), ())),
                       preferred_element_type=jnp.float32)
  acc = acc + lax.dot_general(ph, pp, (((1,), (0,)), ((), ())),
                              preferred_element_type=jnp.float32)
  o_ref[...] = acc


def _project(x4, pidx2, poshead, we3, wp, b2):
  n = x4.shape[0] * 8
  nb = n // M_BLK
  rb = M_BLK // 8
  pdim = wp.shape[1]
  return pl.pallas_call(
      _project_block,
      grid=(nb,),
      in_specs=[
          pl.BlockSpec((rb, NPC, 8, PIECE), lambda i: (i, 0, 0, 0)),
          pl.BlockSpec((1, 1, M_BLK), lambda i: (i, 0, 0)),
          pl.BlockSpec((16, pdim), lambda i: (0, 0)),
          pl.BlockSpec((OUT, NPC, PIECE), lambda i: (0, 0, 0)),
          pl.BlockSpec((OUT, pdim), lambda i: (0, 0)),
          pl.BlockSpec((1, OUT), lambda i: (0, 0)),
      ],
      out_specs=pl.BlockSpec((M_BLK, OUT), lambda i: (i, 0)),
      out_shape=jax.ShapeDtypeStruct((n, OUT), jnp.float32),
      compiler_params=pltpu.CompilerParams(
          dimension_semantics=("arbitrary",)),
  )(x4, pidx2, poshead, we3, wp, b2)


def kernel(entity_ids, entity_table, pos_table, entity_id_to_pos_index, W, b):
  batch_shape = entity_ids.shape
  n = entity_ids.size
  ids = entity_ids.reshape(-1).astype(jnp.int32)
  posmap = entity_id_to_pos_index.astype(jnp.int32)
  idx4 = (ids.reshape(-1, 1, 8) * NPC
          + jnp.arange(NPC, dtype=jnp.int32).reshape(1, NPC, 1)).reshape(-1)
  table4 = entity_table.reshape(-1, PIECE)
  pieces, pidx = _make_sc_gather(n)(table4, posmap, ids, idx4)
  x4 = pieces.reshape(n // 8, NPC, 8, PIECE)
  pidx2 = pidx.reshape(-1, 1, M_BLK)
  we3 = W[:, :EMB].reshape(OUT, NPC, PIECE)
  wp = W[:, EMB:]
  poshead = pos_table[:16]
  out = _project(x4, pidx2, poshead, we3, wp, b.reshape(1, OUT))
  return out.reshape(*batch_shape, OUT)


# speedup vs baseline: 2.6178x; 2.6178x over previous
import functools

import jax
import jax.numpy as jnp
from jax import lax
from jax.experimental import pallas as pl
from jax.experimental.pallas import tpu as pltpu
from jax.experimental.pallas import tpu_sc as plsc

EMB = 512
OUT = 512
NW = 32
CH = 120
M_BLK = 512


def _make_sc_gather(b_total):
  b_per_w = b_total // NW
  nchunk = b_per_w // CH
  assert b_per_w % CH == 0 and b_total % (8 * NW) == 0

  mesh = plsc.VectorSubcoreMesh(core_axis_name="c", subcore_axis_name="s")

  @functools.partial(
      pl.kernel,
      mesh=mesh,
      out_type=[
          jax.ShapeDtypeStruct((b_total, EMB), jnp.float32),
          jax.ShapeDtypeStruct((b_total,), jnp.int32),
      ],
      scratch_types=[
          pltpu.VMEM((b_per_w,), jnp.int32),
          pltpu.VMEM((2, CH, EMB), jnp.float32),
          pltpu.VMEM((b_per_w,), jnp.int32),
          pltpu.SemaphoreType.DMA,
          pltpu.SemaphoreType.DMA,
          pltpu.SemaphoreType.DMA,
          pltpu.SemaphoreType.DMA,
      ],
  )
  def gather_kernel(table_hbm, posmap_hbm, ids_hbm, rows_out_hbm, pos_out_hbm,
                    idx_v, rows_v, pos_v, gsem, psem, osem0, osem1):
    wid = lax.axis_index("s") * 2 + lax.axis_index("c")
    base = wid * b_per_w
    pltpu.sync_copy(ids_hbm.at[pl.ds(base, b_per_w)], idx_v)
    osems = (osem0, osem1)
    pending = [None, None]
    for c in range(nchunk):
      s = c % 2
      if pending[s] is not None:
        pending[s].wait()
      idx_c = idx_v.at[pl.ds(c * CH, CH)]
      gp = pltpu.async_copy(posmap_hbm.at[idx_c],
                            pos_v.at[pl.ds(c * CH, CH)], psem)
      g = pltpu.async_copy(table_hbm.at[idx_c], rows_v.at[s], gsem)
      gp.wait()
      g.wait()
      pending[s] = pltpu.async_copy(
          rows_v.at[s], rows_out_hbm.at[pl.ds(base + c * CH, CH)], osems[s])
    for p in pending:
      if p is not None:
        p.wait()
    pltpu.sync_copy(pos_v, pos_out_hbm.at[pl.ds(base, b_per_w)])

  return gather_kernel


def _project_block(x_ref, pidx_ref, poshead_ref, we_ref, wp_ref, b_ref, o_ref):
  x = x_ref[...]
  pidx = pidx_ref[0, 0, :]
  ph = (pidx[:, None] == lax.broadcasted_iota(jnp.int32, (M_BLK, 16), 1)
        ).astype(jnp.float32)
  pp = lax.dot_general(poshead_ref[...], wp_ref[...],
                       (((1,), (1,)), ((), ())),
                       preferred_element_type=jnp.float32)
  acc = lax.dot_general(x, we_ref[...], (((1,), (1,)), ((), ())),
                        preferred_element_type=jnp.float32)
  acc = acc + lax.dot_general(ph, pp, (((1,), (0,)), ((), ())),
                              preferred_element_type=jnp.float32)
  o_ref[...] = acc + b_ref[...]


def _project(x, pidx3, poshead, we, wp, b2):
  n = x.shape[0]
  nb = n // M_BLK
  pdim = wp.shape[1]
  return pl.pallas_call(
      _project_block,
      grid=(nb,),
      in_specs=[
          pl.BlockSpec((M_BLK, EMB), lambda i: (i, 0)),
          pl.BlockSpec((1, 1, M_BLK), lambda i: (i, 0, 0)),
          pl.BlockSpec((16, pdim), lambda i: (0, 0)),
          pl.BlockSpec((OUT, EMB), lambda i: (0, 0)),
          pl.BlockSpec((OUT, pdim), lambda i: (0, 0)),
          pl.BlockSpec((1, OUT), lambda i: (0, 0)),
      ],
      out_specs=pl.BlockSpec((M_BLK, OUT), lambda i: (i, 0)),
      out_shape=jax.ShapeDtypeStruct((n, OUT), jnp.float32),
      compiler_params=pltpu.CompilerParams(
          dimension_semantics=("arbitrary",)),
  )(x, pidx3, poshead, we, wp, b2)


def kernel(entity_ids, entity_table, pos_table, entity_id_to_pos_index, W, b):
  nb, nc, ne = entity_ids.shape
  n = entity_ids.size
  ids = jnp.transpose(entity_ids, (0, 2, 1)).reshape(-1).astype(jnp.int32)
  posmap = entity_id_to_pos_index.astype(jnp.int32)
  rows, pidx = _make_sc_gather(n)(entity_table, posmap, ids)
  pidx3 = pidx.reshape(n // M_BLK, 1, M_BLK)
  we = W[:, :EMB]
  wp = W[:, EMB:]
  poshead = pos_table[:16]
  out = _project(rows, pidx3, poshead, we, wp, b.reshape(1, OUT))
  return out.reshape(nb, ne, nc, OUT).transpose(0, 2, 1, 3)
